# Initial kernel scaffold; baseline (speedup 1.0000x reference)
import functools

import jax
import jax.numpy as jnp
from jax import lax
from jax.experimental import pallas as pl
from jax.experimental.pallas import tpu as pltpu
from jax.experimental.pallas import tpu_sc as plsc

N = 10000
D = 128
E = 320000
NC = 2
NS = 16
NW = NC * NS
EPW = E // NW
CH = 80
NCHUNK = EPW // CH
STRIPE = N // NS
DEG_W = 16

_mesh = plsc.VectorSubcoreMesh(
    core_axis_name="c", subcore_axis_name="s", num_cores=NC, num_subcores=NS)


@functools.partial(
    pl.kernel,
    out_type=jax.ShapeDtypeStruct((NC, N, DEG_W), jnp.float32),
    mesh=_mesh,
    scratch_types=[
        pltpu.VMEM_SHARED((N, DEG_W), jnp.float32),
        pltpu.VMEM((STRIPE, DEG_W), jnp.float32),
        pltpu.VMEM((CH, DEG_W), jnp.float32),
        pltpu.VMEM((CH,), jnp.int32),
    ],
)
def _sc_degree(dst_hbm, out_hbm, acc, zbuf, ones_v, idx_v):
    c = lax.axis_index("c")
    s = lax.axis_index("s")
    wid = s * NC + c

    def zfill(i, carry):
        zbuf[i, :] = jnp.zeros((DEG_W,), jnp.float32)
        return carry

    lax.fori_loop(0, STRIPE, zfill, 0)

    def ofill(i, carry):
        ones_v[i, :] = jnp.ones((DEG_W,), jnp.float32)
        return carry

    lax.fori_loop(0, CH, ofill, 0)

    r0 = s * STRIPE
    pltpu.sync_copy(zbuf, acc.at[pl.ds(r0, STRIPE)])
    plsc.subcore_barrier()

    def chunk(i, carry):
        base = wid * EPW + i * CH
        pltpu.sync_copy(dst_hbm.at[pl.ds(base, CH)], idx_v)
        pltpu.sync_copy(ones_v, acc.at[idx_v], add=True)
        return carry

    lax.fori_loop(0, NCHUNK, chunk, 0)
    plsc.subcore_barrier()
    pltpu.sync_copy(acc.at[pl.ds(r0, STRIPE)], out_hbm.at[c, pl.ds(r0, STRIPE)])


@functools.partial(
    pl.kernel,
    out_type=jax.ShapeDtypeStruct((NC, N, D), jnp.float32),
    mesh=_mesh,
    scratch_types=[
        pltpu.VMEM_SHARED((N, D), jnp.float32),
        pltpu.VMEM((CH, D), jnp.float32),
        pltpu.VMEM((CH,), jnp.int32),
        pltpu.VMEM((CH,), jnp.int32),
        pltpu.SemaphoreType.DMA,
    ],
)
def _sc_scatter(y_hbm, src_hbm, dst_hbm, out_hbm, acc, rows, sidx, didx, sem):
    c = lax.axis_index("c")
    s = lax.axis_index("s")
    wid = s * NC + c
    r0 = s * STRIPE
    pltpu.sync_copy(y_hbm.at[pl.ds(r0, STRIPE)], acc.at[pl.ds(r0, STRIPE)])
    plsc.subcore_barrier()

    def chunk(i, carry):
        base = wid * EPW + i * CH
        pltpu.sync_copy(src_hbm.at[pl.ds(base, CH)], sidx)
        pltpu.sync_copy(dst_hbm.at[pl.ds(base, CH)], didx)
        pltpu.async_copy(y_hbm.at[sidx], rows, sem).wait()
        pltpu.sync_copy(rows, acc.at[didx], add=True)
        return carry

    lax.fori_loop(0, NCHUNK, chunk, 0)
    plsc.subcore_barrier()
    pltpu.sync_copy(acc.at[pl.ds(r0, STRIPE)], out_hbm.at[c, pl.ds(r0, STRIPE)])


BR = 2000


def _dinv(degp):
    deg = degp[0, :, 0:1] + degp[1, :, 0:1] + 1.0
    return lax.rsqrt(deg)


def _tc_prescale_body(degp_ref, x_ref, w_ref, y_ref):
    y_ref[...] = (
        jnp.dot(x_ref[...], w_ref[...], preferred_element_type=jnp.float32)
        * _dinv(degp_ref[...]))


def _tc_mid_body(degp_ref, p_ref, y1_ref, b_ref, w_ref, y2_ref):
    dinv = _dinv(degp_ref[...])
    p = p_ref[...]
    agg = p[0] + p[1] - y1_ref[...]
    h = jnp.maximum(agg * dinv + b_ref[...], 0.0)
    y2_ref[...] = (
        jnp.dot(h, w_ref[...], preferred_element_type=jnp.float32) * dinv)


def _tc_final_body(degp_ref, p_ref, y2_ref, b_ref, o_ref):
    dinv = _dinv(degp_ref[...])
    p = p_ref[...]
    agg = p[0] + p[1] - y2_ref[...]
    o_ref[...] = agg * dinv + b_ref[...]


_deg_spec = pl.BlockSpec((NC, BR, DEG_W), lambda i: (0, i, 0))
_rows_spec = pl.BlockSpec((BR, D), lambda i: (i, 0))
_p_spec = pl.BlockSpec((NC, BR, D), lambda i: (0, i, 0))
_w_spec = pl.BlockSpec((D, D), lambda i: (0, 0))
_b_spec = pl.BlockSpec((1, D), lambda i: (0, 0))
_GRID = (N // BR,)
_ROWS_OUT = jax.ShapeDtypeStruct((N, D), jnp.float32)

_tc_prescale = pl.pallas_call(
    _tc_prescale_body, grid=_GRID,
    in_specs=[_deg_spec, _rows_spec, _w_spec],
    out_specs=_rows_spec, out_shape=_ROWS_OUT)

_tc_mid = pl.pallas_call(
    _tc_mid_body, grid=_GRID,
    in_specs=[_deg_spec, _p_spec, _rows_spec, _b_spec, _w_spec],
    out_specs=_rows_spec, out_shape=_ROWS_OUT)

_tc_final = pl.pallas_call(
    _tc_final_body, grid=_GRID,
    in_specs=[_deg_spec, _p_spec, _rows_spec, _b_spec],
    out_specs=_rows_spec, out_shape=_ROWS_OUT)


def kernel(x, edge_index, W1, b1, W2, b2):
    src = edge_index[0].astype(jnp.int32)
    dst = edge_index[1].astype(jnp.int32)
    b1r = b1.reshape(1, D)
    b2r = b2.reshape(1, D)

    degp = _sc_degree(dst)
    y1 = _tc_prescale(degp, x, W1)
    p1 = _sc_scatter(y1, src, dst)
    y2 = _tc_mid(degp, p1, y1, b1r, W2)
    p2 = _sc_scatter(y2, src, dst)
    out = _tc_final(degp, p2, y2, b2r)
    return out[:, None, :]


# speedup vs baseline: 12.9975x; 12.9975x over previous
import functools

import jax
import jax.numpy as jnp
from jax import lax
from jax.experimental import pallas as pl
from jax.experimental.pallas import tpu as pltpu
from jax.experimental.pallas import tpu_sc as plsc

N = 10000
NP = 10240
D = 128
E = 320000
NC = 2
NS = 16
NW = NC * NS
EPW = E // NW
CH = 80
NCHUNK = EPW // CH
STRIPE = NP // NS
DEG_W = 16

_mesh = plsc.VectorSubcoreMesh(
    core_axis_name="c", subcore_axis_name="s", num_cores=NC, num_subcores=NS)


@functools.partial(
    pl.kernel,
    out_type=jax.ShapeDtypeStruct((NC, NP, DEG_W), jnp.float32),
    mesh=_mesh,
    scratch_types=[
        pltpu.VMEM_SHARED((NP, DEG_W), jnp.float32),
        pltpu.VMEM((STRIPE, DEG_W), jnp.float32),
        pltpu.VMEM((CH, DEG_W), jnp.float32),
        pltpu.VMEM((CH,), jnp.int32),
    ],
)
def _sc_degree(dst_hbm, out_hbm, acc, zbuf, ones_v, idx_v):
    c = lax.axis_index("c")
    s = lax.axis_index("s")
    wid = s * NC + c

    def zfill(i, carry):
        zbuf[i, :] = jnp.zeros((DEG_W,), jnp.float32)
        return carry

    lax.fori_loop(0, STRIPE, zfill, 0)

    def ofill(i, carry):
        ones_v[i, :] = jnp.ones((DEG_W,), jnp.float32)
        return carry

    lax.fori_loop(0, CH, ofill, 0)

    r0 = s * STRIPE
    pltpu.sync_copy(zbuf, acc.at[pl.ds(r0, STRIPE)])
    plsc.subcore_barrier()

    def chunk(i, carry):
        base = wid * EPW + i * CH
        pltpu.sync_copy(dst_hbm.at[pl.ds(base, CH)], idx_v)
        pltpu.sync_copy(ones_v, acc.at[idx_v], add=True)
        return carry

    lax.fori_loop(0, NCHUNK, chunk, 0)
    plsc.subcore_barrier()
    pltpu.sync_copy(acc.at[pl.ds(r0, STRIPE)], out_hbm.at[c, pl.ds(r0, STRIPE)])


@functools.partial(
    pl.kernel,
    out_type=jax.ShapeDtypeStruct((NC, NP, D), jnp.float32),
    mesh=_mesh,
    scratch_types=[
        pltpu.VMEM_SHARED((NP, D), jnp.float32),
        pltpu.VMEM((CH, D), jnp.float32),
        pltpu.VMEM((CH,), jnp.int32),
        pltpu.VMEM((CH,), jnp.int32),
        pltpu.SemaphoreType.DMA,
    ],
)
def _sc_scatter(y_hbm, src_hbm, dst_hbm, out_hbm, acc, rows, sidx, didx, sem):
    c = lax.axis_index("c")
    s = lax.axis_index("s")
    wid = s * NC + c
    r0 = s * STRIPE
    pltpu.sync_copy(y_hbm.at[pl.ds(r0, STRIPE)], acc.at[pl.ds(r0, STRIPE)])
    plsc.subcore_barrier()

    def chunk(i, carry):
        base = wid * EPW + i * CH
        pltpu.sync_copy(src_hbm.at[pl.ds(base, CH)], sidx)
        pltpu.sync_copy(dst_hbm.at[pl.ds(base, CH)], didx)
        pltpu.async_copy(y_hbm.at[sidx], rows, sem).wait()
        pltpu.sync_copy(rows, acc.at[didx], add=True)
        return carry

    lax.fori_loop(0, NCHUNK, chunk, 0)
    plsc.subcore_barrier()
    pltpu.sync_copy(acc.at[pl.ds(r0, STRIPE)], out_hbm.at[c, pl.ds(r0, STRIPE)])


BR = 2048


def _dinv(degp):
    deg = degp[0, :, 0:1] + degp[1, :, 0:1] + 1.0
    return lax.rsqrt(deg)


def _tc_prescale_body(degp_ref, x_ref, w_ref, y_ref):
    y_ref[...] = (
        jnp.dot(x_ref[...], w_ref[...], preferred_element_type=jnp.float32)
        * _dinv(degp_ref[...]))


def _tc_mid_body(degp_ref, p_ref, y1_ref, b_ref, w_ref, y2_ref):
    dinv = _dinv(degp_ref[...])
    p = p_ref[...]
    agg = p[0] + p[1] - y1_ref[...]
    h = jnp.maximum(agg * dinv + b_ref[...], 0.0)
    y2_ref[...] = (
        jnp.dot(h, w_ref[...], preferred_element_type=jnp.float32) * dinv)


def _tc_final_body(degp_ref, p_ref, y2_ref, b_ref, o_ref):
    dinv = _dinv(degp_ref[...])
    p = p_ref[...]
    agg = p[0] + p[1] - y2_ref[...]
    o_ref[...] = agg * dinv + b_ref[...]


_deg_spec = pl.BlockSpec((NC, BR, DEG_W), lambda i: (0, i, 0))
_rows_spec = pl.BlockSpec((BR, D), lambda i: (i, 0))
_p_spec = pl.BlockSpec((NC, BR, D), lambda i: (0, i, 0))
_w_spec = pl.BlockSpec((D, D), lambda i: (0, 0))
_b_spec = pl.BlockSpec((1, D), lambda i: (0, 0))
_GRID = (NP // BR,)
_ROWS_OUT = jax.ShapeDtypeStruct((NP, D), jnp.float32)

_tc_prescale = pl.pallas_call(
    _tc_prescale_body, grid=_GRID,
    in_specs=[_deg_spec, _rows_spec, _w_spec],
    out_specs=_rows_spec, out_shape=_ROWS_OUT)

_tc_mid = pl.pallas_call(
    _tc_mid_body, grid=_GRID,
    in_specs=[_deg_spec, _p_spec, _rows_spec, _b_spec, _w_spec],
    out_specs=_rows_spec, out_shape=_ROWS_OUT)

_tc_final = pl.pallas_call(
    _tc_final_body, grid=_GRID,
    in_specs=[_deg_spec, _p_spec, _rows_spec, _b_spec],
    out_specs=_rows_spec, out_shape=_ROWS_OUT)


def kernel(x, edge_index, W1, b1, W2, b2):
    src = edge_index[0].astype(jnp.int32)
    dst = edge_index[1].astype(jnp.int32)
    b1r = b1.reshape(1, D)
    b2r = b2.reshape(1, D)

    xp = jnp.pad(x, ((0, NP - N), (0, 0)))
    degp = _sc_degree(dst)
    y1 = _tc_prescale(degp, xp, W1)
    p1 = _sc_scatter(y1, src, dst)
    y2 = _tc_mid(degp, p1, y1, b1r, W2)
    p2 = _sc_scatter(y2, src, dst)
    out = _tc_final(degp, p2, y2, b2r)
    return out[:N, None, :]


# speedup vs baseline: 28.1244x; 2.1638x over previous
import functools

import jax
import jax.numpy as jnp
from jax import lax
from jax.experimental import pallas as pl
from jax.experimental.pallas import tpu as pltpu
from jax.experimental.pallas import tpu_sc as plsc

N = 10000
NP = 10240
D = 128
E = 320000
NC = 2
NS = 16
NW = NC * NS
EPW = E // NW
CH = 80
NCHUNK = EPW // CH
STRIPE = NP // NS
DEG_W = 16
K = 4
NOUT = NCHUNK // K
NREM = NCHUNK - NOUT * K

_mesh = plsc.VectorSubcoreMesh(
    core_axis_name="c", subcore_axis_name="s", num_cores=NC, num_subcores=NS)


@functools.partial(
    pl.kernel,
    out_type=jax.ShapeDtypeStruct((NC, NP, DEG_W), jnp.float32),
    mesh=_mesh,
    scratch_types=[
        pltpu.VMEM_SHARED((NP, DEG_W), jnp.float32),
        pltpu.VMEM((STRIPE, DEG_W), jnp.float32),
        pltpu.VMEM((CH, DEG_W), jnp.float32),
        pltpu.VMEM((NCHUNK, CH), jnp.int32),
        pltpu.SemaphoreType.DMA,
    ],
)
def _sc_degree(dst_hbm, out_hbm, acc, zbuf, ones_v, didx, ssem):
    c = lax.axis_index("c")
    s = lax.axis_index("s")
    wid = s * NC + c

    def zfill(i, carry):
        zbuf[i, :] = jnp.zeros((DEG_W,), jnp.float32)
        return carry

    lax.fori_loop(0, STRIPE, zfill, 0)

    def ofill(i, carry):
        ones_v[i, :] = jnp.ones((DEG_W,), jnp.float32)
        return carry

    lax.fori_loop(0, CH, ofill, 0)

    pltpu.sync_copy(dst_hbm.at[wid], didx)
    r0 = s * STRIPE
    pltpu.sync_copy(zbuf, acc.at[pl.ds(r0, STRIPE)])
    plsc.subcore_barrier()

    def outer(g, carry):
        descs = [
            pltpu.async_copy(ones_v, acc.at[didx.at[g * K + b]], ssem, add=True)
            for b in range(K)
        ]
        for d in descs:
            d.wait()
        return carry

    lax.fori_loop(0, NCHUNK // K, outer, 0)
    for j in range(NCHUNK - (NCHUNK // K) * K):
        pltpu.async_copy(
            ones_v, acc.at[didx.at[(NCHUNK // K) * K + j]], ssem,
            add=True).wait()
    plsc.subcore_barrier()
    pltpu.sync_copy(acc.at[pl.ds(r0, STRIPE)], out_hbm.at[c, pl.ds(r0, STRIPE)])


_ROWS_T = pltpu.VMEM((CH, D), jnp.float32)
_IDX_T = pltpu.VMEM((CH,), jnp.int32)


@functools.partial(
    pl.kernel,
    out_type=jax.ShapeDtypeStruct((NC, NP, D), jnp.float32),
    mesh=_mesh,
    scratch_types=(
        [pltpu.VMEM_SHARED((NP, D), jnp.float32)]
        + [_ROWS_T] * K + [_IDX_T] * K + [_IDX_T] * K
        + [pltpu.SemaphoreType.DMA] * 3
    ),
)
def _sc_scatter(y_hbm, src_hbm, dst_hbm, out_hbm, acc, *scr):
    rows = scr[:K]
    sidx = scr[K:2 * K]
    didx = scr[2 * K:3 * K]
    isem, gsem, ssem = scr[3 * K:]
    c = lax.axis_index("c")
    s = lax.axis_index("s")
    wid = s * NC + c
    r0 = s * STRIPE
    pltpu.sync_copy(y_hbm.at[pl.ds(r0, STRIPE)], acc.at[pl.ds(r0, STRIPE)])
    plsc.subcore_barrier()

    def outer(g, carry):
        j0 = g * K
        ids = []
        for b in range(K):
            ids.append(pltpu.async_copy(src_hbm.at[wid, j0 + b, 0], sidx[b], isem))
            ids.append(pltpu.async_copy(dst_hbm.at[wid, j0 + b, 0], didx[b], isem))
        gds = []
        for b in range(K):
            ids[2 * b].wait()
            gds.append(pltpu.async_copy(y_hbm.at[sidx[b]], rows[b], gsem))
        sds = []
        for b in range(K):
            gds[b].wait()
            ids[2 * b + 1].wait()
            sds.append(pltpu.async_copy(rows[b], acc.at[didx[b]], ssem,
                                        add=True))
        for d in sds:
            d.wait()
        return carry

    lax.fori_loop(0, NOUT, outer, 0)
    for j in range(NREM):
        jc = NOUT * K + j
        pltpu.sync_copy(src_hbm.at[wid, jc, 0], sidx[0])
        pltpu.sync_copy(dst_hbm.at[wid, jc, 0], didx[0])
        pltpu.async_copy(y_hbm.at[sidx[0]], rows[0], gsem).wait()
        pltpu.sync_copy(rows[0], acc.at[didx[0]], add=True)
    plsc.subcore_barrier()
    pltpu.sync_copy(acc.at[pl.ds(r0, STRIPE)], out_hbm.at[c, pl.ds(r0, STRIPE)])


BR = 2048


def _dinv(degp):
    deg = degp[0, :, 0:1] + degp[1, :, 0:1] + 1.0
    return lax.rsqrt(deg)


def _tc_prescale_body(degp_ref, x_ref, w_ref, y_ref):
    y_ref[...] = (
        jnp.dot(x_ref[...], w_ref[...], preferred_element_type=jnp.float32)
        * _dinv(degp_ref[...]))


def _tc_mid_body(degp_ref, p_ref, y1_ref, b_ref, w_ref, y2_ref):
    dinv = _dinv(degp_ref[...])
    p = p_ref[...]
    agg = p[0] + p[1] - y1_ref[...]
    h = jnp.maximum(agg * dinv + b_ref[...], 0.0)
    y2_ref[...] = (
        jnp.dot(h, w_ref[...], preferred_element_type=jnp.float32) * dinv)


def _tc_final_body(degp_ref, p_ref, y2_ref, b_ref, o_ref):
    dinv = _dinv(degp_ref[...])
    p = p_ref[...]
    agg = p[0] + p[1] - y2_ref[...]
    o_ref[...] = agg * dinv + b_ref[...]


_deg_spec = pl.BlockSpec((NC, BR, DEG_W), lambda i: (0, i, 0))
_rows_spec = pl.BlockSpec((BR, D), lambda i: (i, 0))
_p_spec = pl.BlockSpec((NC, BR, D), lambda i: (0, i, 0))
_w_spec = pl.BlockSpec((D, D), lambda i: (0, 0))
_b_spec = pl.BlockSpec((1, D), lambda i: (0, 0))
_GRID = (NP // BR,)
_ROWS_OUT = jax.ShapeDtypeStruct((NP, D), jnp.float32)

_tc_prescale = pl.pallas_call(
    _tc_prescale_body, grid=_GRID,
    in_specs=[_deg_spec, _rows_spec, _w_spec],
    out_specs=_rows_spec, out_shape=_ROWS_OUT)

_tc_mid = pl.pallas_call(
    _tc_mid_body, grid=_GRID,
    in_specs=[_deg_spec, _p_spec, _rows_spec, _b_spec, _w_spec],
    out_specs=_rows_spec, out_shape=_ROWS_OUT)

_tc_final = pl.pallas_call(
    _tc_final_body, grid=_GRID,
    in_specs=[_deg_spec, _p_spec, _rows_spec, _b_spec],
    out_specs=_rows_spec, out_shape=_ROWS_OUT)


def kernel(x, edge_index, W1, b1, W2, b2):
    src = edge_index[0].astype(jnp.int32).reshape(NW, NCHUNK, 1, CH)
    dst = edge_index[1].astype(jnp.int32).reshape(NW, NCHUNK, 1, CH)
    b1r = b1.reshape(1, D)
    b2r = b2.reshape(1, D)

    xp = jnp.pad(x, ((0, NP - N), (0, 0)))
    degp = _sc_degree(dst.reshape(NW, NCHUNK, CH))
    y1 = _tc_prescale(degp, xp, W1)
    p1 = _sc_scatter(y1, src, dst)
    y2 = _tc_mid(degp, p1, y1, b1r, W2)
    p2 = _sc_scatter(y2, src, dst)
    out = _tc_final(degp, p2, y2, b2r)
    return out[:N, None, :]
